# baseline (device time: 124344 ns/iter reference)
import jax
import jax.numpy as jnp
from jax import lax
from jax.experimental import pallas as pl
from jax.experimental.pallas import tpu as pltpu

N_DEV = 8
M_PER = 512
HALF = 256


def kernel(x, w_mat, scale_x, scale_w):
    m_per, k = x.shape
    _, n_per = w_mat.shape
    scale = (scale_x[0] * scale_w[0]).reshape(1, 1)

    def body(x_ref, w_ref, scale_ref, out_ref,
             fwd_ref, bwd_ref, fwd_send, fwd_recv, bwd_send, bwd_recv):
        my = lax.axis_index("i")
        right = (my + 1) % N_DEV
        left = (my + N_DEV - 1) % N_DEV

        barrier = pltpu.get_barrier_semaphore()
        for nbr in (left, right):
            pl.semaphore_signal(barrier, inc=1, device_id=(nbr,),
                                device_id_type=pl.DeviceIdType.MESH)
        pl.semaphore_wait(barrier, 2)

        def gemm(chunk, row0):
            acc = lax.dot_general(
                chunk.astype(jnp.bfloat16), w_ref[...].astype(jnp.bfloat16),
                dimension_numbers=(((1,), (0,)), ((), ())),
                preferred_element_type=jnp.float32,
            )
            out_ref[pl.ds(row0, chunk.shape[0]), :] = acc * scale_ref[0, 0]

        fwd_ref[my] = x_ref[0:HALF, :]
        bwd_ref[my] = x_ref[HALF:M_PER, :]

        def start_sends(h):
            of_s = (my + N_DEV - h) % N_DEV
            ob_s = (my + h) % N_DEV
            pltpu.make_async_remote_copy(
                src_ref=fwd_ref.at[of_s], dst_ref=fwd_ref.at[of_s],
                send_sem=fwd_send.at[h], recv_sem=fwd_recv.at[h],
                device_id=(right,), device_id_type=pl.DeviceIdType.MESH,
            ).start()
            pltpu.make_async_remote_copy(
                src_ref=bwd_ref.at[ob_s], dst_ref=bwd_ref.at[ob_s],
                send_sem=bwd_send.at[h], recv_sem=bwd_recv.at[h],
                device_id=(left,), device_id_type=pl.DeviceIdType.MESH,
            ).start()

        start_sends(0)
        gemm(x_ref[...], my * M_PER)

        for h in range(N_DEV - 1):
            of_r = (my + N_DEV - h - 1) % N_DEV
            ob_r = (my + h + 1) % N_DEV

            pltpu.make_async_remote_copy(
                src_ref=fwd_ref.at[of_r], dst_ref=fwd_ref.at[of_r],
                send_sem=fwd_send.at[h], recv_sem=fwd_recv.at[h],
                device_id=(right,), device_id_type=pl.DeviceIdType.MESH,
            ).wait_recv()
            pltpu.make_async_remote_copy(
                src_ref=bwd_ref.at[ob_r], dst_ref=bwd_ref.at[ob_r],
                send_sem=bwd_send.at[h], recv_sem=bwd_recv.at[h],
                device_id=(left,), device_id_type=pl.DeviceIdType.MESH,
            ).wait_recv()

            if h < N_DEV - 2:
                start_sends(h + 1)

            gemm(fwd_ref[of_r], of_r * M_PER)
            gemm(bwd_ref[ob_r], ob_r * M_PER + HALF)

        for h in range(N_DEV - 1):
            of_s = (my + N_DEV - h) % N_DEV
            ob_s = (my + h) % N_DEV
            pltpu.make_async_remote_copy(
                src_ref=fwd_ref.at[of_s], dst_ref=fwd_ref.at[of_s],
                send_sem=fwd_send.at[h], recv_sem=fwd_recv.at[h],
                device_id=(right,), device_id_type=pl.DeviceIdType.MESH,
            ).wait_send()
            pltpu.make_async_remote_copy(
                src_ref=bwd_ref.at[ob_s], dst_ref=bwd_ref.at[ob_s],
                send_sem=bwd_send.at[h], recv_sem=bwd_recv.at[h],
                device_id=(left,), device_id_type=pl.DeviceIdType.MESH,
            ).wait_send()

    return pl.pallas_call(
        body,
        out_shape=jax.ShapeDtypeStruct((N_DEV * m_per, n_per), jnp.float32),
        in_specs=[
            pl.BlockSpec(memory_space=pltpu.VMEM),
            pl.BlockSpec(memory_space=pltpu.VMEM),
            pl.BlockSpec(memory_space=pltpu.SMEM),
        ],
        out_specs=pl.BlockSpec(memory_space=pltpu.VMEM),
        scratch_shapes=[
            pltpu.VMEM((N_DEV, HALF, k), jnp.int8),
            pltpu.VMEM((N_DEV, HALF, k), jnp.int8),
            pltpu.SemaphoreType.DMA((N_DEV - 1,)),
            pltpu.SemaphoreType.DMA((N_DEV - 1,)),
            pltpu.SemaphoreType.DMA((N_DEV - 1,)),
            pltpu.SemaphoreType.DMA((N_DEV - 1,)),
        ],
        compiler_params=pltpu.CompilerParams(
            collective_id=0, vmem_limit_bytes=100 * 1024 * 1024,
        ),
    )(x, w_mat, scale)


# device time: 111769 ns/iter; 1.1125x vs baseline; 1.1125x over previous
import jax
import jax.numpy as jnp
from jax import lax
from jax.experimental import pallas as pl
from jax.experimental.pallas import tpu as pltpu

N_DEV = 8
M_PER = 512
HALF = 256
NSUB = 2
SUB = HALF // NSUB


def kernel(x, w_mat, scale_x, scale_w):
    m_per, k = x.shape
    _, n_per = w_mat.shape
    scale = (scale_x[0] * scale_w[0]).reshape(1, 1)

    def body(x_ref, w_ref, scale_ref, out_ref,
             fwd_ref, bwd_ref, fwd_send, fwd_recv, bwd_send, bwd_recv):
        my = lax.axis_index("i")
        right = (my + 1) % N_DEV
        left = (my + N_DEV - 1) % N_DEV

        barrier = pltpu.get_barrier_semaphore()
        for nbr in (left, right):
            pl.semaphore_signal(barrier, inc=1, device_id=(nbr,),
                                device_id_type=pl.DeviceIdType.MESH)
        pl.semaphore_wait(barrier, 2)

        def gemm(chunk, row0):
            acc = lax.dot_general(
                chunk.astype(jnp.bfloat16), w_ref[...].astype(jnp.bfloat16),
                dimension_numbers=(((1,), (0,)), ((), ())),
                preferred_element_type=jnp.float32,
            )
            out_ref[pl.ds(row0, chunk.shape[0]), :] = acc * scale_ref[0, 0]

        def rdma(buf, origin, s, send_sems, recv_sems, h, dst):
            return pltpu.make_async_remote_copy(
                src_ref=buf.at[origin, s], dst_ref=buf.at[origin, s],
                send_sem=send_sems.at[h, s], recv_sem=recv_sems.at[h, s],
                device_id=(dst,), device_id_type=pl.DeviceIdType.MESH,
            )

        fwd_ref[my] = x_ref[0:HALF, :].reshape(NSUB, SUB, k)
        bwd_ref[my] = x_ref[HALF:M_PER, :].reshape(NSUB, SUB, k)

        def of_send(h):
            return (my + N_DEV - h) % N_DEV

        def ob_send(h):
            return (my + h) % N_DEV

        for s in range(NSUB):
            rdma(fwd_ref, of_send(0), s, fwd_send, fwd_recv, 0, right).start()
            rdma(bwd_ref, ob_send(0), s, bwd_send, bwd_recv, 0, left).start()
        gemm(x_ref[...], my * M_PER)

        for h in range(N_DEV - 1):
            of_r = (my + N_DEV - h - 1) % N_DEV
            ob_r = (my + h + 1) % N_DEV

            for s in range(NSUB):
                rdma(fwd_ref, of_r, s, fwd_send, fwd_recv, h, right).wait_recv()
                if h < N_DEV - 2:
                    rdma(fwd_ref, of_r, s, fwd_send, fwd_recv, h + 1,
                         right).start()
                rdma(bwd_ref, ob_r, s, bwd_send, bwd_recv, h, left).wait_recv()
                if h < N_DEV - 2:
                    rdma(bwd_ref, ob_r, s, bwd_send, bwd_recv, h + 1,
                         left).start()

            gemm(fwd_ref[of_r].reshape(HALF, k), of_r * M_PER)
            gemm(bwd_ref[ob_r].reshape(HALF, k), ob_r * M_PER + HALF)

        for h in range(N_DEV - 1):
            for s in range(NSUB):
                rdma(fwd_ref, of_send(h), s, fwd_send, fwd_recv, h,
                     right).wait_send()
                rdma(bwd_ref, ob_send(h), s, bwd_send, bwd_recv, h,
                     left).wait_send()

    return pl.pallas_call(
        body,
        out_shape=jax.ShapeDtypeStruct((N_DEV * m_per, n_per), jnp.float32),
        in_specs=[
            pl.BlockSpec(memory_space=pltpu.VMEM),
            pl.BlockSpec(memory_space=pltpu.VMEM),
            pl.BlockSpec(memory_space=pltpu.SMEM),
        ],
        out_specs=pl.BlockSpec(memory_space=pltpu.VMEM),
        scratch_shapes=[
            pltpu.VMEM((N_DEV, NSUB, SUB, k), jnp.int8),
            pltpu.VMEM((N_DEV, NSUB, SUB, k), jnp.int8),
            pltpu.SemaphoreType.DMA((N_DEV - 1, NSUB)),
            pltpu.SemaphoreType.DMA((N_DEV - 1, NSUB)),
            pltpu.SemaphoreType.DMA((N_DEV - 1, NSUB)),
            pltpu.SemaphoreType.DMA((N_DEV - 1, NSUB)),
        ],
        compiler_params=pltpu.CompilerParams(
            collective_id=0, vmem_limit_bytes=100 * 1024 * 1024,
        ),
    )(x, w_mat, scale)
